# baseline (speedup 1.0000x reference)
import functools

import jax
import jax.numpy as jnp
from jax import lax
from jax.experimental import pallas as pl
from jax.experimental.pallas import tpu as pltpu
from jax.experimental.pallas import tpu_sc as plsc

EMBED_DIM = 32
NUM_CORES = 2
NUM_SUBCORES = 16
NUM_WORKERS = NUM_CORES * NUM_SUBCORES
HCH = 4


@functools.partial(jax.jit, static_argnums=(2, 3))
def _gather_sc(idx_t, table, bw, n_steps):
    mesh = plsc.VectorSubcoreMesh(core_axis_name="c", subcore_axis_name="s")
    hist, batch = idx_t.shape
    rows_per_step = HCH * bw

    @functools.partial(
        pl.kernel,
        mesh=mesh,
        out_type=jax.ShapeDtypeStruct((hist, EMBED_DIM, batch), jnp.float32),
        scratch_types=[
            pltpu.VMEM((HCH, bw), jnp.int32),
            pltpu.VMEM((HCH, bw), jnp.int32),
            pltpu.VMEM((rows_per_step, EMBED_DIM), jnp.float32),
            pltpu.VMEM((rows_per_step, EMBED_DIM), jnp.float32),
            pltpu.VMEM((HCH, EMBED_DIM, bw), jnp.float32),
            pltpu.VMEM((HCH, EMBED_DIM, bw), jnp.float32),
            pltpu.SemaphoreType.DMA,
            pltpu.SemaphoreType.DMA,
            pltpu.SemaphoreType.DMA,
            pltpu.SemaphoreType.DMA,
            pltpu.SemaphoreType.DMA,
            pltpu.SemaphoreType.DMA,
        ],
        compiler_params=pltpu.CompilerParams(
            use_tc_tiling_on_sc=False, needs_layout_passes=False),
    )
    def k(idx_hbm, table_hbm, out_hbm, ib0, ib1, wide0, wide1, tb0, tb1,
          i0, i1, g0, g1, o0, o1):
        ibuf = (ib0, ib1)
        wide = (wide0, wide1)
        tbuf = (tb0, tb1)
        isem = (i0, i1)
        gsem = (g0, g1)
        osem = (o0, o1)
        wid = lax.axis_index("s") * NUM_CORES + lax.axis_index("c")
        b0 = wid * bw
        lanes = lax.iota(jnp.int32, 16)

        def idx_load(s, b):
            pltpu.async_copy(
                idx_hbm.at[pl.ds(s * HCH, HCH), pl.ds(b0, bw)],
                ibuf[b], isem[b])

        def idx_wait(b):
            pltpu.make_async_copy(
                idx_hbm.at[pl.ds(0, HCH), pl.ds(0, bw)], ibuf[b],
                isem[b]).wait()

        def gather_start(b):
            for hh in range(HCH):
                pltpu.async_copy(
                    table_hbm.at[ibuf[b].at[hh]],
                    wide[b].at[pl.ds(hh * bw, bw)], gsem[b])

        def gather_wait(b):
            pltpu.make_async_copy(
                table_hbm.at[pl.ds(0, rows_per_step)], wide[b],
                gsem[b]).wait()

        def transpose(b):
            for hh in range(HCH):
                for q in range(bw // 16):
                    rowv = lanes + (hh * bw + q * 16)
                    for d in range(EMBED_DIM):
                        colv = jnp.full((16,), d, jnp.int32)
                        v = plsc.load_gather(wide[b], [rowv, colv])
                        tbuf[b][hh, d, pl.ds(q * 16, 16)] = v

        def store_start(s, b):
            pltpu.async_copy(
                tbuf[b],
                out_hbm.at[pl.ds(s * HCH, HCH), :, pl.ds(b0, bw)],
                osem[b])

        def store_wait(b):
            pltpu.make_async_copy(
                tbuf[b], out_hbm.at[pl.ds(0, HCH), :, pl.ds(0, bw)],
                osem[b]).wait()

        idx_load(0, 0)
        idx_wait(0)
        gather_start(0)
        idx_load(1, 1)

        def body(g, carry):
            for j in range(2):
                s = 2 * g + j
                b = j
                gather_wait(b)

                @pl.when(s < n_steps - 1)
                def _():
                    idx_wait(1 - b)
                    gather_start(1 - b)

                @pl.when(s < n_steps - 2)
                def _():
                    idx_load(s + 2, b)

                @pl.when(s >= 2)
                def _():
                    store_wait(b)

                transpose(b)
                store_start(s, b)
            return carry

        lax.fori_loop(0, n_steps // 2, body, 0)
        store_wait(0)
        store_wait(1)

    return k(idx_t, table)


def kernel(input, table):
    batch, hist = input.shape
    bw = batch // NUM_WORKERS
    n_steps = hist // HCH
    idx_t = input.T.astype(jnp.int32)
    out_t = _gather_sc(idx_t, table, bw, n_steps)
    return jnp.transpose(out_t, (2, 0, 1))


# speedup vs baseline: 2.1142x; 2.1142x over previous
import functools

import jax
import jax.numpy as jnp
from jax import lax
from jax.experimental import pallas as pl
from jax.experimental.pallas import tpu as pltpu
from jax.experimental.pallas import tpu_sc as plsc

EMBED_DIM = 32
NUM_CORES = 2
NUM_SUBCORES = 16
NUM_WORKERS = NUM_CORES * NUM_SUBCORES
HCH = 4
DPAD = 133


@functools.partial(jax.jit, static_argnums=(2, 3))
def _gather_sc(idx_t, table, bw, n_steps):
    mesh = plsc.VectorSubcoreMesh(core_axis_name="c", subcore_axis_name="s")
    hist, batch = idx_t.shape
    rows_per_step = HCH * bw

    @functools.partial(
        pl.kernel,
        mesh=mesh,
        out_type=jax.ShapeDtypeStruct(
            (hist, EMBED_DIM // 8, batch // 128, 8, 128), jnp.float32),
        scratch_types=[
            pltpu.VMEM((HCH, bw), jnp.int32),
            pltpu.VMEM((HCH, bw), jnp.int32),
            pltpu.VMEM((rows_per_step, EMBED_DIM), jnp.float32),
            pltpu.VMEM((rows_per_step, EMBED_DIM), jnp.float32),
            pltpu.VMEM((HCH, EMBED_DIM // 8, 1, 8, DPAD), jnp.float32),
            pltpu.VMEM((HCH, EMBED_DIM // 8, 1, 8, DPAD), jnp.float32),
            pltpu.SemaphoreType.DMA,
            pltpu.SemaphoreType.DMA,
            pltpu.SemaphoreType.DMA,
            pltpu.SemaphoreType.DMA,
            pltpu.SemaphoreType.DMA,
            pltpu.SemaphoreType.DMA,
        ],
        compiler_params=pltpu.CompilerParams(
            use_tc_tiling_on_sc=False, needs_layout_passes=False),
    )
    def k(idx_hbm, table_hbm, out_hbm, ib0, ib1, wide0, wide1, tb0, tb1,
          i0, i1, g0, g1, o0, o1):
        ibuf = (ib0, ib1)
        wide = (wide0, wide1)
        tbuf = (tb0, tb1)
        isem = (i0, i1)
        gsem = (g0, g1)
        osem = (o0, o1)
        wid = lax.axis_index("s") * NUM_CORES + lax.axis_index("c")
        b0 = wid * bw
        lanes = lax.iota(jnp.int32, 16)
        zeros16 = jnp.zeros((16,), jnp.int32)
        d8_lo = jnp.right_shift(lanes, 3)
        dr_lo = lanes & 7

        def idx_load(s, b):
            pltpu.async_copy(
                idx_hbm.at[pl.ds(s * HCH, HCH), pl.ds(b0, bw)],
                ibuf[b], isem[b])

        def idx_wait(b):
            pltpu.make_async_copy(
                idx_hbm.at[pl.ds(0, HCH), pl.ds(0, bw)], ibuf[b],
                isem[b]).wait()

        def gather_start(b):
            for hh in range(HCH):
                pltpu.async_copy(
                    table_hbm.at[ibuf[b].at[hh]],
                    wide[b].at[pl.ds(hh * bw, bw)], gsem[b])

        def gather_wait(b):
            pltpu.make_async_copy(
                table_hbm.at[pl.ds(0, rows_per_step)], wide[b],
                gsem[b]).wait()

        def transpose(b):
            for hh in range(HCH):
                hh_v = jnp.full((16,), hh, jnp.int32)

                def tr_body(j, carry):
                    row = hh * bw + j
                    jv = jnp.full((16,), j, jnp.int32)
                    v0 = wide[b][row, 0:16]
                    v1 = wide[b][row, 16:32]
                    plsc.store_scatter(
                        tbuf[b], [hh_v, d8_lo, zeros16, dr_lo, jv], v0)
                    plsc.store_scatter(
                        tbuf[b], [hh_v, d8_lo + 2, zeros16, dr_lo, jv], v1)
                    return carry

                lax.fori_loop(0, bw, tr_body, 0, unroll=8)

        def store_start(s, b):
            pltpu.async_copy(
                tbuf[b].at[:, :, :, :, pl.ds(0, bw)],
                out_hbm.at[pl.ds(s * HCH, HCH), :, pl.ds(wid, 1), :, :],
                osem[b])

        def store_wait(b):
            pltpu.make_async_copy(
                tbuf[b].at[:, :, :, :, pl.ds(0, bw)],
                out_hbm.at[pl.ds(0, HCH), :, pl.ds(0, 1), :, :],
                osem[b]).wait()

        idx_load(0, 0)
        idx_wait(0)
        gather_start(0)
        idx_load(1, 1)

        def body(g, carry):
            for j in range(2):
                s = 2 * g + j
                b = j
                gather_wait(b)

                @pl.when(s < n_steps - 1)
                def _():
                    idx_wait(1 - b)
                    gather_start(1 - b)

                @pl.when(s < n_steps - 2)
                def _():
                    idx_load(s + 2, b)

                @pl.when(s >= 2)
                def _():
                    store_wait(b)

                transpose(b)
                store_start(s, b)
            return carry

        lax.fori_loop(0, n_steps // 2, body, 0)
        store_wait(0)
        store_wait(1)

    return k(idx_t, table)


def kernel(input, table):
    batch, hist = input.shape
    bw = batch // NUM_WORKERS
    n_steps = hist // HCH
    idx_t = input.T.astype(jnp.int32)
    out5 = _gather_sc(idx_t, table, bw, n_steps)
    return jnp.transpose(out5, (2, 4, 0, 1, 3)).reshape(batch, hist,
                                                        EMBED_DIM)
